# Initial kernel scaffold; baseline (speedup 1.0000x reference)
import jax
import jax.numpy as jnp
from jax.experimental import pallas as pl


def kernel(seq_ids, W, b):
    raise NotImplementedError("write your pallas kernel here")


# speedup vs baseline: 1.5585x; 1.5585x over previous
import functools

import jax
import jax.numpy as jnp
from jax import lax
from jax.experimental import pallas as pl
from jax.experimental.pallas import tpu as pltpu
from jax.experimental.pallas import tpu_sc as plsc

EMBED = 128
VOCAB = 21
NC, NS = 2, 16
NW = NC * NS
CHUNK = 128


def _sc_lookup(table, idx, n, per_w, n_chunks):
    mesh = plsc.VectorSubcoreMesh(core_axis_name="c", subcore_axis_name="s")

    @functools.partial(
        pl.kernel,
        out_type=jax.ShapeDtypeStruct((n, EMBED), jnp.float32),
        mesh=mesh,
        scratch_types=[
            pltpu.VMEM((n_chunks, CHUNK), jnp.int32),
            pltpu.VMEM((CHUNK, EMBED), jnp.float32),
            pltpu.SemaphoreType.DMA,
        ],
    )
    def k(table_hbm, idx_hbm, out_hbm, idx_v, rows_v, sem):
        wid = lax.axis_index("s") * NC + lax.axis_index("c")
        base = wid * per_w
        pltpu.sync_copy(idx_hbm.at[wid], idx_v)

        def body(j, carry):
            pltpu.async_copy(table_hbm.at[idx_v.at[j]], rows_v, sem).wait()
            pltpu.sync_copy(rows_v, out_hbm.at[pl.ds(base + j * CHUNK, CHUNK)])
            return carry

        lax.fori_loop(0, n_chunks, body, 0)

    return k(table, idx)


def kernel(seq_ids, W, b):
    B, L = seq_ids.shape
    n = B * L
    per_w = n // NW
    n_chunks = per_w // CHUNK
    table = W.T + b
    idx = seq_ids.reshape(NW, n_chunks, CHUNK).astype(jnp.int32)
    out = _sc_lookup(table, idx, n, per_w, n_chunks)
    return out.reshape(B, L, EMBED)


# speedup vs baseline: 1.5791x; 1.0132x over previous
import functools

import jax
import jax.numpy as jnp
from jax import lax
from jax.experimental import pallas as pl
from jax.experimental.pallas import tpu as pltpu
from jax.experimental.pallas import tpu_sc as plsc

EMBED = 128
VOCAB = 21
NC, NS = 2, 16
NW = NC * NS
CHUNK = 128
M = 2


def _sc_lookup(table, idx, n_chunks):
    groups = n_chunks // M
    half = groups // 2
    mesh = plsc.VectorSubcoreMesh(core_axis_name="c", subcore_axis_name="s")

    @functools.partial(
        pl.kernel,
        out_type=jax.ShapeDtypeStruct((NW * groups, M, CHUNK, EMBED), jnp.float32),
        mesh=mesh,
        scratch_types=[
            pltpu.VMEM((n_chunks, CHUNK), jnp.int32),
            pltpu.VMEM((2, M, CHUNK, EMBED), jnp.float32),
            pltpu.SemaphoreType.DMA,
            pltpu.SemaphoreType.DMA,
            pltpu.SemaphoreType.DMA,
            pltpu.SemaphoreType.DMA,
        ],
    )
    def k(table_hbm, idx_hbm, out_hbm, idx_v, rows_v, g0, g1, s0, s1):
        gsem = (g0, g1)
        ssem = (s0, s1)
        wid = lax.axis_index("s") * NC + lax.axis_index("c")
        gbase = wid * groups
        pltpu.sync_copy(idx_hbm.at[wid], idx_v)

        def fire(slot, g):
            for i in range(M):
                pltpu.async_copy(table_hbm.at[idx_v.at[g * M + i]],
                                 rows_v.at[slot, i], gsem[slot])

        def drain(slot, g):
            for i in range(M):
                pltpu.make_async_copy(table_hbm.at[idx_v.at[g * M + i]],
                                      rows_v.at[slot, i], gsem[slot]).wait()

        def scat(slot, g):
            pltpu.async_copy(rows_v.at[slot], out_hbm.at[gbase + g], ssem[slot])

        def scat_wait(slot, g):
            pltpu.make_async_copy(rows_v.at[slot], out_hbm.at[gbase + g],
                                  ssem[slot]).wait()

        fire(0, 0)

        def body(h, carry):
            ga = 2 * h
            gb = 2 * h + 1

            @pl.when(h >= 1)
            def _():
                scat_wait(1, gb - 2)

            fire(1, gb)
            drain(0, ga)
            scat(0, ga)
            drain(1, gb)
            scat(1, gb)
            scat_wait(0, ga)

            @pl.when(h + 1 < half)
            def _():
                fire(0, ga + 2)

            return carry

        lax.fori_loop(0, half, body, 0)
        scat_wait(1, 2 * half - 1)

    return k(table, idx)


def kernel(seq_ids, W, b):
    B, L = seq_ids.shape
    n = B * L
    per_w = n // NW
    n_chunks = per_w // CHUNK
    table = W.T + b
    idx = seq_ids.reshape(NW, n_chunks, CHUNK).astype(jnp.int32)
    out = _sc_lookup(table, idx, n_chunks)
    return out.reshape(B, L, EMBED)


# speedup vs baseline: 4.5907x; 2.9071x over previous
import functools

import jax
import jax.numpy as jnp
import numpy as np
from jax import lax
from jax.experimental import pallas as pl
from jax.experimental.pallas import tpu as pltpu
from jax.experimental.pallas import tpu_sc as plsc

EMBED = 128
VOCAB = 21
NC, NS = 2, 16
NW = NC * NS
SLOT = 256
RG = 16


def _sc_lookup(w_flat, b, idx, per_w):
    groups = per_w // SLOT
    half = groups // 2
    mesh = plsc.VectorSubcoreMesh(core_axis_name="c", subcore_axis_name="s")

    @functools.partial(
        pl.kernel,
        out_type=jax.ShapeDtypeStruct((NW * groups, SLOT, EMBED), jnp.float32),
        mesh=mesh,
        compiler_params=pltpu.CompilerParams(needs_layout_passes=False),
        scratch_types=[
            pltpu.VMEM((per_w,), jnp.int32),
            pltpu.VMEM((VOCAB * EMBED,), jnp.float32),
            pltpu.VMEM((EMBED,), jnp.float32),
            pltpu.VMEM((VOCAB * EMBED,), jnp.float32),
            pltpu.VMEM((2, SLOT, EMBED), jnp.float32),
            pltpu.SemaphoreType.DMA,
            pltpu.SemaphoreType.DMA,
        ],
    )
    def k(w_hbm, b_hbm, idx_hbm, out_hbm, idx_v, w_v, b_v, tab_v, rows_v,
          s0, s1):
        ssem = (s0, s1)
        iota = lax.iota(jnp.int32, 16)
        wid = lax.axis_index("s") * NC + lax.axis_index("c")
        gbase = wid * groups
        pltpu.sync_copy(idx_hbm.at[wid], idx_v)
        pltpu.sync_copy(w_hbm, w_v)
        pltpu.sync_copy(b_hbm, b_v)

        bvecs = [b_v[pl.ds(e8 * 16, 16)] for e8 in range(8)]
        for v in range(VOCAB):
            for e8 in range(8):
                widx = (e8 * 16 + iota) * VOCAB + v
                col = plsc.load_gather(w_v, [widx])
                tab_v[pl.ds(v * EMBED + e8 * 16, 16)] = col + bvecs[e8]

        def expand(slot, g):
            def body(rb, carry):
                r0 = g * SLOT + rb * RG
                idxv = idx_v[pl.ds(r0, RG)]
                for rr in range(RG):
                    base = idxv[rr] * EMBED + iota
                    for e8 in range(8):
                        col = plsc.load_gather(tab_v, [base + e8 * 16])
                        rows_v[slot, rb * RG + rr, pl.ds(e8 * 16, 16)] = col
                return carry

            lax.fori_loop(0, SLOT // RG, body, 0)

        def scat(slot, g):
            pltpu.async_copy(rows_v.at[slot], out_hbm.at[gbase + g], ssem[slot])

        def scat_wait(slot, g):
            pltpu.make_async_copy(rows_v.at[slot], out_hbm.at[gbase + g],
                                  ssem[slot]).wait()

        def body(h, carry):
            ga = 2 * h
            gb = 2 * h + 1

            @pl.when(h >= 1)
            def _():
                scat_wait(0, ga - 2)

            expand(0, ga)
            scat(0, ga)

            @pl.when(h >= 1)
            def _():
                scat_wait(1, gb - 2)

            expand(1, gb)
            scat(1, gb)
            return carry

        lax.fori_loop(0, half, body, 0)
        scat_wait(0, 2 * half - 2)
        scat_wait(1, 2 * half - 1)

    return k(w_flat, b, idx)


def kernel(seq_ids, W, b):
    B, L = seq_ids.shape
    n = B * L
    per_w = n // NW
    idx = seq_ids.reshape(NW, per_w).astype(jnp.int32)
    out = _sc_lookup(W.reshape(-1), b, idx, per_w)
    return out.reshape(B, L, EMBED)
